# Initial kernel scaffold; baseline (speedup 1.0000x reference)
import jax
import jax.numpy as jnp
from jax.experimental import pallas as pl


def kernel(x, frequency_embedding, phase_embedding):
    raise NotImplementedError("write your pallas kernel here")


# speedup vs baseline: 4.2395x; 4.2395x over previous
import functools

import jax
import jax.numpy as jnp
from jax import lax
from jax.experimental import pallas as pl
from jax.experimental.pallas import tpu as pltpu
from jax.experimental.pallas import tpu_sc as plsc

_NC = 2
_NS = 16
_LANES = 16


def _make_sc_kernel(N, V, D, L, interpret=False):
    NW = _NC * _NS
    assert N % (NW * L) == 0, (N, NW, L)
    RW = N // NW
    C = L
    NG = RW // C
    JD = D // _LANES

    mesh = plsc.VectorSubcoreMesh(
        core_axis_name="c", subcore_axis_name="s",
        num_cores=_NC, num_subcores=_NS)

    def body(x_hbm, tab_hbm, freq_hbm, out_hbm, idx_all, freqv, pos, buf, sem):
        wid = lax.axis_index("s") * _NC + lax.axis_index("c")
        base = wid * RW
        pltpu.sync_copy(x_hbm.at[pl.ds(base, RW)], idx_all)
        pltpu.sync_copy(freq_hbm.at[0], freqv)

        def build_pos(r, carry):
            rf = jnp.float32(r)
            for j in range(JD):
                pos[r, pl.ds(j * _LANES, _LANES)] = (
                    rf * freqv[pl.ds(j * _LANES, _LANES)])
            return carry
        lax.fori_loop(0, L, build_pos, 0)

        def chunk(g, carry):
            row0 = base + g * C
            pltpu.async_copy(
                tab_hbm.at[idx_all.at[pl.ds(g * C, C)]], buf, sem).wait()

            def comp(r, c2):
                for j in range(JD):
                    sl = pl.ds(j * _LANES, _LANES)
                    v = buf[r, sl]
                    buf[r, sl] = pos[r, sl] + 6.28 / (1.0 + jnp.exp(-v))
                return c2
            lax.fori_loop(0, C, comp, 0)
            pltpu.sync_copy(buf, out_hbm.at[pl.ds(row0, C)])
            return carry
        lax.fori_loop(0, NG, chunk, 0)

    return pl.kernel(
        body,
        out_type=jax.ShapeDtypeStruct((N, D), jnp.float32),
        mesh=mesh,
        scratch_types=[
            pltpu.VMEM((RW,), jnp.int32),
            pltpu.VMEM((D,), jnp.float32),
            pltpu.VMEM((L, D), jnp.float32),
            pltpu.VMEM((C, D), jnp.float32),
            pltpu.SemaphoreType.DMA,
        ],
        compiler_params=pltpu.CompilerParams(use_tc_tiling_on_sc=False),
        interpret=interpret,
    )


@jax.jit
def kernel(x, frequency_embedding, phase_embedding):
    B, L = x.shape
    V, D = phase_embedding.shape
    N = B * L
    sc = _make_sc_kernel(N, V, D, L)
    out = sc(x.reshape(N), phase_embedding, frequency_embedding)
    return out.reshape(B, L, D)


# speedup vs baseline: 7.2141x; 1.7016x over previous
import functools

import jax
import jax.numpy as jnp
from jax import lax
from jax.experimental import pallas as pl
from jax.experimental.pallas import tpu as pltpu
from jax.experimental.pallas import tpu_sc as plsc

_NC = 2
_NS = 16
_LANES = 16


def _make_sc_kernel(N, V, D, L):
    NW = _NC * _NS
    assert N % (NW * L) == 0, (N, NW, L)
    RW = N // NW
    C = L
    NG = RW // C
    JD = D // _LANES
    assert NG >= 4 and NG % 2 == 0

    mesh = plsc.VectorSubcoreMesh(
        core_axis_name="c", subcore_axis_name="s",
        num_cores=_NC, num_subcores=_NS)

    def body(x_hbm, tab_hbm, freq_hbm, out_hbm, idx_all, freqv, pos,
             gb0, gb1, ob0, ob1, gs0, gs1, os0, os1):
        wid = lax.axis_index("s") * _NC + lax.axis_index("c")
        base = wid * RW
        pltpu.sync_copy(x_hbm.at[pl.ds(base, RW)], idx_all)
        pltpu.sync_copy(freq_hbm.at[0], freqv)

        def build_pos(r, carry):
            rf = jnp.float32(r)
            for j in range(JD):
                pos[r, pl.ds(j * _LANES, _LANES)] = (
                    rf * freqv[pl.ds(j * _LANES, _LANES)])
            return carry
        lax.fori_loop(0, L, build_pos, 0)

        gb = (gb0, gb1)
        ob = (ob0, ob1)
        gs = (gs0, gs1)
        os_ = (os0, os1)

        def start_gather(g, b):
            pltpu.async_copy(
                tab_hbm.at[idx_all.at[pl.ds(g * C, C)]], gb[b], gs[b])

        def wait_gather(b):
            pltpu.make_async_copy(
                out_hbm.at[pl.ds(0, C)], gb[b], gs[b]).wait()

        def start_out(g, b):
            pltpu.async_copy(
                ob[b], out_hbm.at[pl.ds(base + g * C, C)], os_[b])

        def wait_out(b):
            pltpu.make_async_copy(
                ob[b], out_hbm.at[pl.ds(0, C)], os_[b]).wait()

        def compute(b):
            gbuf, obuf = gb[b], ob[b]

            def comp(r, c2):
                for j in range(JD):
                    sl = pl.ds(j * _LANES, _LANES)
                    v = gbuf[r, sl]
                    obuf[r, sl] = pos[r, sl] + 6.28 / (1.0 + jnp.exp(-v))
                return c2
            lax.fori_loop(0, C, comp, 0)

        start_gather(0, 0)
        start_gather(1, 1)
        for b in (0, 1):
            wait_gather(b)
            compute(b)
            start_gather(b + 2, b)
            start_out(b, b)

        def main(g2, carry):
            g0 = 2 * g2
            for b in (0, 1):
                wait_gather(b)
                wait_out(b)
                compute(b)
                start_gather(g0 + b + 2, b)
                start_out(g0 + b, b)
            return carry
        lax.fori_loop(1, NG // 2 - 1, main, 0)

        for b in (0, 1):
            wait_gather(b)
            wait_out(b)
            compute(b)
            start_out(NG - 2 + b, b)
        wait_out(0)
        wait_out(1)

    return pl.kernel(
        body,
        out_type=jax.ShapeDtypeStruct((N, D), jnp.float32),
        mesh=mesh,
        scratch_types=[
            pltpu.VMEM((RW,), jnp.int32),
            pltpu.VMEM((D,), jnp.float32),
            pltpu.VMEM((L, D), jnp.float32),
            pltpu.VMEM((C, D), jnp.float32),
            pltpu.VMEM((C, D), jnp.float32),
            pltpu.VMEM((C, D), jnp.float32),
            pltpu.VMEM((C, D), jnp.float32),
            pltpu.SemaphoreType.DMA,
            pltpu.SemaphoreType.DMA,
            pltpu.SemaphoreType.DMA,
            pltpu.SemaphoreType.DMA,
        ],
        compiler_params=pltpu.CompilerParams(use_tc_tiling_on_sc=False),
    )


@jax.jit
def kernel(x, frequency_embedding, phase_embedding):
    B, L = x.shape
    V, D = phase_embedding.shape
    N = B * L
    sc = _make_sc_kernel(N, V, D, L)
    out = sc(x.reshape(N), phase_embedding, frequency_embedding)
    return out.reshape(B, L, D)


# speedup vs baseline: 7.2757x; 1.0085x over previous
import functools

import jax
import jax.numpy as jnp
from jax import lax
from jax.experimental import pallas as pl
from jax.experimental.pallas import tpu as pltpu
from jax.experimental.pallas import tpu_sc as plsc

_NC = 2
_NS = 16
_LANES = 16


def _make_sc_kernel(N, V, D, L):
    NW = _NC * _NS
    assert N % (NW * L) == 0, (N, NW, L)
    RW = N // NW
    C = L
    NG = RW // C
    JD = D // _LANES
    assert NG >= 4 and NG % 2 == 0

    mesh = plsc.VectorSubcoreMesh(
        core_axis_name="c", subcore_axis_name="s",
        num_cores=_NC, num_subcores=_NS)

    def body(x_hbm, tab_hbm, freq_hbm, out_hbm, idx_all, freqv, pos,
             gb0, gb1, ob0, ob1, gs0, gs1, os0, os1):
        wid = lax.axis_index("s") * _NC + lax.axis_index("c")
        base = wid * RW
        pltpu.sync_copy(x_hbm.at[pl.ds(base, RW)], idx_all)
        pltpu.sync_copy(freq_hbm.at[0], freqv)

        def build_pos(r, carry):
            rf = jnp.float32(r)
            for j in range(JD):
                pos[r, pl.ds(j * _LANES, _LANES)] = (
                    rf * freqv[pl.ds(j * _LANES, _LANES)])
            return carry
        lax.fori_loop(0, L, build_pos, 0)

        gb = (gb0, gb1)
        ob = (ob0, ob1)
        gs = (gs0, gs1)
        os_ = (os0, os1)

        def start_gather(g, b):
            pltpu.async_copy(
                tab_hbm.at[idx_all.at[pl.ds(g * C, C)]], gb[b], gs[b])

        def wait_gather(b):
            pltpu.make_async_copy(
                out_hbm.at[pl.ds(0, C)], gb[b], gs[b]).wait()

        def start_out(g, b):
            pltpu.async_copy(
                ob[b], out_hbm.at[pl.ds(base + g * C, C)], os_[b])

        def wait_out(b):
            pltpu.make_async_copy(
                ob[b], out_hbm.at[pl.ds(0, C)], os_[b]).wait()

        def compute(b):
            gbuf, obuf = gb[b], ob[b]

            @plsc.parallel_loop(0, C, step=1, unroll=2)
            def comp(r):
                for j in range(JD):
                    sl = pl.ds(j * _LANES, _LANES)
                    v = gbuf[r, sl]
                    obuf[r, sl] = pos[r, sl] + 6.28 / (1.0 + jnp.exp(-v))

        start_gather(0, 0)
        start_gather(1, 1)
        for b in (0, 1):
            wait_gather(b)
            compute(b)
            start_gather(b + 2, b)
            start_out(b, b)

        def main(g2, carry):
            g0 = 2 * g2
            for b in (0, 1):
                wait_gather(b)
                wait_out(b)
                compute(b)
                start_gather(g0 + b + 2, b)
                start_out(g0 + b, b)
            return carry
        lax.fori_loop(1, NG // 2 - 1, main, 0)

        for b in (0, 1):
            wait_gather(b)
            wait_out(b)
            compute(b)
            start_out(NG - 2 + b, b)
        wait_out(0)
        wait_out(1)

    return pl.kernel(
        body,
        out_type=jax.ShapeDtypeStruct((N, D), jnp.float32),
        mesh=mesh,
        scratch_types=[
            pltpu.VMEM((RW,), jnp.int32),
            pltpu.VMEM((D,), jnp.float32),
            pltpu.VMEM((L, D), jnp.float32),
            pltpu.VMEM((C, D), jnp.float32),
            pltpu.VMEM((C, D), jnp.float32),
            pltpu.VMEM((C, D), jnp.float32),
            pltpu.VMEM((C, D), jnp.float32),
            pltpu.SemaphoreType.DMA,
            pltpu.SemaphoreType.DMA,
            pltpu.SemaphoreType.DMA,
            pltpu.SemaphoreType.DMA,
        ],
        compiler_params=pltpu.CompilerParams(use_tc_tiling_on_sc=False),
    )


@jax.jit
def kernel(x, frequency_embedding, phase_embedding):
    B, L = x.shape
    V, D = phase_embedding.shape
    N = B * L
    sc = _make_sc_kernel(N, V, D, L)
    out = sc(x.reshape(N), phase_embedding, frequency_embedding)
    return out.reshape(B, L, D)
